# baseline (speedup 1.0000x reference)
import functools

import jax
import jax.numpy as jnp
from jax import lax
from jax.experimental import pallas as pl
from jax.experimental.pallas import tpu as pltpu
from jax.experimental.pallas import tpu_sc as plsc

N = 16384
K = 2048

NSC = 5120
NTC = N - NSC

NSUB = 32
RP = NSC // NSUB
CH = K // 16
G = 8

BNP = 1024
BNT = 1024


def _prep_body(pt_ref, th_ref, c_ref, rd_ref, cb_ref):
    pt = pt_ref[...]
    th = th_ref[...]
    cos = jnp.cos(th)
    sin = jnp.sin(th)
    dx1 = pt[2:3, :] - pt[0:1, :]
    dy1 = pt[3:4, :] - pt[1:2, :]
    dx2 = pt[4:5, :] - pt[0:1, :]
    dy2 = pt[5:6, :] - pt[1:2, :]
    px1 = dx1 * cos + dy1 * sin
    py1 = dy1 * cos - dx1 * sin
    px2 = dx2 * cos + dy2 * sin
    py2 = dy2 * cos - dx2 * sin
    pn = dx1 * dx1 + dy1 * dy1 + dx2 * dx2 + dy2 * dy2
    rd_ref[...] = jnp.concatenate([px1, py1, px2, py2, pn], axis=0)

    c = c_ref[...]
    cx1 = c[2:3, :]
    cy1 = c[3:4, :]
    cx2 = c[4:5, :]
    cy2 = c[5:6, :]
    e = (c[0:1, :] * c[0:1, :] + c[1:2, :] * c[1:2, :]
         + cx1 * cx1 + cy1 * cy1 + cx2 * cx2 + cy2 * cy2)
    zk3 = jnp.zeros((3, K), jnp.float32)
    cb_ref[...] = jnp.concatenate(
        [2.0 * cx1, 2.0 * cy1, 2.0 * cx2, 2.0 * cy2, e, zk3], axis=0)


def _tc_prep(pt_sc, th_sc, c):
    return pl.pallas_call(
        _prep_body,
        grid=(NSC // BNP,),
        in_specs=[
            pl.BlockSpec((6, BNP), lambda i: (0, i)),
            pl.BlockSpec((1, BNP), lambda i: (0, i)),
            pl.BlockSpec((6, K), lambda i: (0, 0)),
        ],
        out_specs=[
            pl.BlockSpec((5, BNP), lambda i: (0, i)),
            pl.BlockSpec((8, K), lambda i: (0, 0)),
        ],
        out_shape=[
            jax.ShapeDtypeStruct((5, NSC), jnp.float32),
            jax.ShapeDtypeStruct((8, K), jnp.float32),
        ],
    )(pt_sc, th_sc, c)


def _sc_body(cb_hbm, rd_hbm, idx_hbm, md_hbm, cb_v, rd_v, idx_v, md_v):
    wid = lax.axis_index("s") * 2 + lax.axis_index("c")
    base = wid * RP
    pltpu.sync_copy(cb_hbm, cb_v)
    for comp in range(5):
        pltpu.sync_copy(rd_hbm.at[pl.ds(comp * NSC + base, RP)],
                        rd_v.at[pl.ds(comp * RP, RP)])
    kiota = lax.iota(jnp.int32, 16)
    lane0 = kiota == 0

    def macro_body(mb, _):
        rbase = mb * 16
        av1 = rd_v[pl.ds(0 * RP + rbase, 16)]
        av2 = rd_v[pl.ds(1 * RP + rbase, 16)]
        av3 = rd_v[pl.ds(2 * RP + rbase, 16)]
        av4 = rd_v[pl.ds(3 * RP + rbase, 16)]
        apn = rd_v[pl.ds(4 * RP + rbase, 16)]

        for sub in range(16 // G):
            s1, s2, s3, s4 = [], [], [], []
            for i in range(G):
                li = jnp.full((16,), sub * G + i, jnp.int32)
                s1.append(jnp.take_along_axis(av1, li, axis=0))
                s2.append(jnp.take_along_axis(av2, li, axis=0))
                s3.append(jnp.take_along_axis(av3, li, axis=0))
                s4.append(jnp.take_along_axis(av4, li, axis=0))

            def chunk_body(j, carry, s1=s1, s2=s2, s3=s3, s4=s4):
                best, bidx = carry
                o = j * 16
                c1 = cb_v[0, pl.ds(o, 16)]
                c2 = cb_v[1, pl.ds(o, 16)]
                c3 = cb_v[2, pl.ds(o, 16)]
                c4 = cb_v[3, pl.ds(o, 16)]
                ev = cb_v[4, pl.ds(o, 16)]
                jv = jnp.full((16,), j, jnp.int32)
                nbest, nbidx = [], []
                for i in range(G):
                    d = ev - (c1 * s1[i] + c2 * s2[i] + c3 * s3[i] + c4 * s4[i])
                    lt = d < best[i]
                    nbest.append(jnp.where(lt, d, best[i]))
                    nbidx.append(jnp.where(lt, jv, bidx[i]))
                return tuple(nbest), tuple(nbidx)

            best0 = tuple(jnp.full((16,), jnp.inf, jnp.float32) for _ in range(G))
            bidx0 = tuple(jnp.zeros((16,), jnp.int32) for _ in range(G))
            best, bidx = lax.fori_loop(0, CH, chunk_body, (best0, bidx0))

            for i in range(G):
                mv = jnp.min(best[i])
                bi = jnp.min(jnp.where(best[i] == mv,
                                       bidx[i] * 16 + kiota, jnp.int32(K)))
                r = rbase + sub * G + i
                rv = jnp.full((16,), r, jnp.int32)
                plsc.store_scatter(idx_v, [rv], jnp.full((16,), bi, jnp.int32),
                                   mask=lane0)
                plsc.store_scatter(md_v, [rv], jnp.full((16,), mv + apn[sub * G + i],
                                                        jnp.float32), mask=lane0)
        return 0

    lax.fori_loop(0, RP // 16, macro_body, 0)
    pltpu.sync_copy(idx_v, idx_hbm.at[pl.ds(base, RP)])
    pltpu.sync_copy(md_v, md_hbm.at[pl.ds(base, RP)])


_sc_call = functools.partial(
    pl.kernel,
    mesh=plsc.VectorSubcoreMesh(core_axis_name="c", subcore_axis_name="s"),
    compiler_params=pltpu.CompilerParams(needs_layout_passes=False),
    out_type=[
        jax.ShapeDtypeStruct((NSC,), jnp.int32),
        jax.ShapeDtypeStruct((NSC,), jnp.float32),
    ],
    scratch_types=[
        pltpu.VMEM((8, K), jnp.float32),
        pltpu.VMEM((5 * RP,), jnp.float32),
        pltpu.VMEM((RP,), jnp.int32),
        pltpu.VMEM((RP,), jnp.float32),
    ],
)(_sc_body)


KB = 256


def _tc_match_body(pt_ref, th_ref, ct_ref, idx_ref, md_ref):
    pt = pt_ref[...]
    th = th_ref[...]
    cos = jnp.cos(th)
    sin = jnp.sin(th)
    dx1 = pt[2:3, :] - pt[0:1, :]
    dy1 = pt[3:4, :] - pt[1:2, :]
    dx2 = pt[4:5, :] - pt[0:1, :]
    dy2 = pt[5:6, :] - pt[1:2, :]
    px1 = dx1 * cos + dy1 * sin
    py1 = dy1 * cos - dx1 * sin
    px2 = dx2 * cos + dy2 * sin
    py2 = dy2 * cos - dx2 * sin
    pn = dx1 * dx1 + dy1 * dy1 + dx2 * dx2 + dy2 * dy2

    ct = ct_ref[...]
    e_all = jnp.sum(ct * ct, axis=1, keepdims=True)

    m = jnp.full((1, BNT), jnp.inf, jnp.float32)
    am = jnp.full((1, BNT), K, jnp.int32)
    iota0 = lax.broadcasted_iota(jnp.int32, (KB, BNT), 0)
    for b in range(K // KB):
        sl = slice(b * KB, (b + 1) * KB)
        cx1 = ct[sl, 2:3]
        cy1 = ct[sl, 3:4]
        cx2 = ct[sl, 4:5]
        cy2 = ct[sl, 5:6]
        d = e_all[sl] - ((cx1 + cx1) * px1 + (cy1 + cy1) * py1
                         + (cx2 + cx2) * px2 + (cy2 + cy2) * py2)
        mc = jnp.min(d, axis=0, keepdims=True)
        cand = jnp.where(d <= mc, iota0, jnp.int32(K))
        amc = jnp.min(cand, axis=0, keepdims=True) + b * KB
        upd = mc < m
        m = jnp.where(upd, mc, m)
        am = jnp.where(upd, amc, am)

    idx_ref[...] = am
    md_ref[...] = m + pn


def _tc_match(pt_tc, th_tc, ct):
    nb = NTC // BNT
    idx2, md2 = pl.pallas_call(
        _tc_match_body,
        grid=(nb,),
        in_specs=[
            pl.BlockSpec((6, BNT), lambda i: (0, i)),
            pl.BlockSpec((1, BNT), lambda i: (0, i)),
            pl.BlockSpec((K, 6), lambda i: (0, 0)),
        ],
        out_specs=[
            pl.BlockSpec((1, BNT), lambda i: (0, i)),
            pl.BlockSpec((1, BNT), lambda i: (0, i)),
        ],
        out_shape=[
            jax.ShapeDtypeStruct((1, NTC), jnp.int32),
            jax.ShapeDtypeStruct((1, NTC), jnp.float32),
        ],
    )(pt_tc, th_tc, ct)
    return idx2.reshape(NTC), md2.reshape(NTC)


@jax.jit
def kernel(traj_pos, traj_theta, map_token_sample_pt):
    p6 = traj_pos.reshape(N, 6).T
    th = traj_theta.reshape(1, N)
    c = map_token_sample_pt.reshape(K, 6).T

    rd, cb = _tc_prep(p6[:, NTC:], th[:, NTC:], c)
    idx_sc, md_sc = _sc_call(cb, rd.reshape(5 * NSC))

    idx_tc, md_tc = _tc_match(p6[:, :NTC], th[:, :NTC],
                              map_token_sample_pt.reshape(K, 6))

    idx = jnp.concatenate([idx_tc, idx_sc])
    md = jnp.concatenate([md_tc, md_sc])
    return (traj_pos[:, 0], traj_theta, idx, md)


# speedup vs baseline: 1.0035x; 1.0035x over previous
import functools

import jax
import jax.numpy as jnp
from jax import lax
from jax.experimental import pallas as pl
from jax.experimental.pallas import tpu as pltpu
from jax.experimental.pallas import tpu_sc as plsc

N = 16384
K = 2048

NSC = 5120
NTC = N - NSC

NSUB = 32
RP = NSC // NSUB
CH = K // 16
G = 4

BNP = 1024
BNT = 1024


def _prep_body(pt_ref, th_ref, c_ref, rd_ref, cb_ref):
    pt = pt_ref[...]
    th = th_ref[...]
    cos = jnp.cos(th)
    sin = jnp.sin(th)
    dx1 = pt[2:3, :] - pt[0:1, :]
    dy1 = pt[3:4, :] - pt[1:2, :]
    dx2 = pt[4:5, :] - pt[0:1, :]
    dy2 = pt[5:6, :] - pt[1:2, :]
    px1 = dx1 * cos + dy1 * sin
    py1 = dy1 * cos - dx1 * sin
    px2 = dx2 * cos + dy2 * sin
    py2 = dy2 * cos - dx2 * sin
    pn = dx1 * dx1 + dy1 * dy1 + dx2 * dx2 + dy2 * dy2
    rd_ref[...] = jnp.concatenate([px1, py1, px2, py2, pn], axis=0)

    c = c_ref[...]
    cx1 = c[2:3, :]
    cy1 = c[3:4, :]
    cx2 = c[4:5, :]
    cy2 = c[5:6, :]
    e = (c[0:1, :] * c[0:1, :] + c[1:2, :] * c[1:2, :]
         + cx1 * cx1 + cy1 * cy1 + cx2 * cx2 + cy2 * cy2)
    zk3 = jnp.zeros((3, K), jnp.float32)
    cb_ref[...] = jnp.concatenate(
        [2.0 * cx1, 2.0 * cy1, 2.0 * cx2, 2.0 * cy2, e, zk3], axis=0)


def _tc_prep(pt_sc, th_sc, c):
    return pl.pallas_call(
        _prep_body,
        grid=(NSC // BNP,),
        in_specs=[
            pl.BlockSpec((6, BNP), lambda i: (0, i)),
            pl.BlockSpec((1, BNP), lambda i: (0, i)),
            pl.BlockSpec((6, K), lambda i: (0, 0)),
        ],
        out_specs=[
            pl.BlockSpec((5, BNP), lambda i: (0, i)),
            pl.BlockSpec((8, K), lambda i: (0, 0)),
        ],
        out_shape=[
            jax.ShapeDtypeStruct((5, NSC), jnp.float32),
            jax.ShapeDtypeStruct((8, K), jnp.float32),
        ],
    )(pt_sc, th_sc, c)


def _sc_body(cb_hbm, rd_hbm, idx_hbm, md_hbm, cb_v, rd_v, idx_v, md_v):
    wid = lax.axis_index("s") * 2 + lax.axis_index("c")
    base = wid * RP
    pltpu.sync_copy(cb_hbm, cb_v)
    for comp in range(5):
        pltpu.sync_copy(rd_hbm.at[pl.ds(comp * NSC + base, RP)],
                        rd_v.at[pl.ds(comp * RP, RP)])
    kiota = lax.iota(jnp.int32, 16)
    lane0 = kiota == 0

    def macro_body(mb, _):
        rbase = mb * 16
        av1 = rd_v[pl.ds(0 * RP + rbase, 16)]
        av2 = rd_v[pl.ds(1 * RP + rbase, 16)]
        av3 = rd_v[pl.ds(2 * RP + rbase, 16)]
        av4 = rd_v[pl.ds(3 * RP + rbase, 16)]
        apn = rd_v[pl.ds(4 * RP + rbase, 16)]

        for sub in range(16 // G):
            s1, s2, s3, s4 = [], [], [], []
            for i in range(G):
                li = jnp.full((16,), sub * G + i, jnp.int32)
                s1.append(jnp.take_along_axis(av1, li, axis=0))
                s2.append(jnp.take_along_axis(av2, li, axis=0))
                s3.append(jnp.take_along_axis(av3, li, axis=0))
                s4.append(jnp.take_along_axis(av4, li, axis=0))

            def chunk_body(j, carry, s1=s1, s2=s2, s3=s3, s4=s4):
                best, bidx = carry
                o = j * 16
                c1 = cb_v[0, pl.ds(o, 16)]
                c2 = cb_v[1, pl.ds(o, 16)]
                c3 = cb_v[2, pl.ds(o, 16)]
                c4 = cb_v[3, pl.ds(o, 16)]
                ev = cb_v[4, pl.ds(o, 16)]
                jv = jnp.full((16,), j, jnp.int32)
                nbest, nbidx = [], []
                for i in range(G):
                    d = ev - (c1 * s1[i] + c2 * s2[i] + c3 * s3[i] + c4 * s4[i])
                    lt = d < best[i]
                    nbest.append(jnp.where(lt, d, best[i]))
                    nbidx.append(jnp.where(lt, jv, bidx[i]))
                return tuple(nbest), tuple(nbidx)

            best0 = tuple(jnp.full((16,), jnp.inf, jnp.float32) for _ in range(G))
            bidx0 = tuple(jnp.zeros((16,), jnp.int32) for _ in range(G))
            best, bidx = lax.fori_loop(0, CH, chunk_body, (best0, bidx0))

            for i in range(G):
                mv = jnp.min(best[i])
                bi = jnp.min(jnp.where(best[i] == mv,
                                       bidx[i] * 16 + kiota, jnp.int32(K)))
                r = rbase + sub * G + i
                rv = jnp.full((16,), r, jnp.int32)
                plsc.store_scatter(idx_v, [rv], jnp.full((16,), bi, jnp.int32),
                                   mask=lane0)
                plsc.store_scatter(md_v, [rv], jnp.full((16,), mv + apn[sub * G + i],
                                                        jnp.float32), mask=lane0)
        return 0

    lax.fori_loop(0, RP // 16, macro_body, 0)
    pltpu.sync_copy(idx_v, idx_hbm.at[pl.ds(base, RP)])
    pltpu.sync_copy(md_v, md_hbm.at[pl.ds(base, RP)])


_sc_call = functools.partial(
    pl.kernel,
    mesh=plsc.VectorSubcoreMesh(core_axis_name="c", subcore_axis_name="s"),
    compiler_params=pltpu.CompilerParams(needs_layout_passes=False),
    out_type=[
        jax.ShapeDtypeStruct((NSC,), jnp.int32),
        jax.ShapeDtypeStruct((NSC,), jnp.float32),
    ],
    scratch_types=[
        pltpu.VMEM((8, K), jnp.float32),
        pltpu.VMEM((5 * RP,), jnp.float32),
        pltpu.VMEM((RP,), jnp.int32),
        pltpu.VMEM((RP,), jnp.float32),
    ],
)(_sc_body)


KB = 512


def _tc_match_body(pt_ref, th_ref, ct_ref, idx_ref, md_ref):
    pt = pt_ref[...]
    th = th_ref[...]
    cos = jnp.cos(th)
    sin = jnp.sin(th)
    dx1 = pt[2:3, :] - pt[0:1, :]
    dy1 = pt[3:4, :] - pt[1:2, :]
    dx2 = pt[4:5, :] - pt[0:1, :]
    dy2 = pt[5:6, :] - pt[1:2, :]
    px1 = dx1 * cos + dy1 * sin
    py1 = dy1 * cos - dx1 * sin
    px2 = dx2 * cos + dy2 * sin
    py2 = dy2 * cos - dx2 * sin
    pn = dx1 * dx1 + dy1 * dy1 + dx2 * dx2 + dy2 * dy2

    ct = ct_ref[...]
    e_all = jnp.sum(ct * ct, axis=1, keepdims=True)

    m = jnp.full((1, BNT), jnp.inf, jnp.float32)
    am = jnp.full((1, BNT), K, jnp.int32)
    iota0 = lax.broadcasted_iota(jnp.int32, (KB, BNT), 0)
    for b in range(K // KB):
        sl = slice(b * KB, (b + 1) * KB)
        cx1 = ct[sl, 2:3]
        cy1 = ct[sl, 3:4]
        cx2 = ct[sl, 4:5]
        cy2 = ct[sl, 5:6]
        d = e_all[sl] - ((cx1 + cx1) * px1 + (cy1 + cy1) * py1
                         + (cx2 + cx2) * px2 + (cy2 + cy2) * py2)
        mc = jnp.min(d, axis=0, keepdims=True)
        cand = jnp.where(d <= mc, iota0, jnp.int32(K))
        amc = jnp.min(cand, axis=0, keepdims=True) + b * KB
        upd = mc < m
        m = jnp.where(upd, mc, m)
        am = jnp.where(upd, amc, am)

    idx_ref[...] = am
    md_ref[...] = m + pn


def _tc_match(pt_tc, th_tc, ct):
    nb = NTC // BNT
    idx2, md2 = pl.pallas_call(
        _tc_match_body,
        grid=(nb,),
        in_specs=[
            pl.BlockSpec((6, BNT), lambda i: (0, i)),
            pl.BlockSpec((1, BNT), lambda i: (0, i)),
            pl.BlockSpec((K, 6), lambda i: (0, 0)),
        ],
        out_specs=[
            pl.BlockSpec((1, BNT), lambda i: (0, i)),
            pl.BlockSpec((1, BNT), lambda i: (0, i)),
        ],
        out_shape=[
            jax.ShapeDtypeStruct((1, NTC), jnp.int32),
            jax.ShapeDtypeStruct((1, NTC), jnp.float32),
        ],
    )(pt_tc, th_tc, ct)
    return idx2.reshape(NTC), md2.reshape(NTC)


@jax.jit
def kernel(traj_pos, traj_theta, map_token_sample_pt):
    p6 = traj_pos.reshape(N, 6).T
    th = traj_theta.reshape(1, N)
    c = map_token_sample_pt.reshape(K, 6).T

    rd, cb = _tc_prep(p6[:, NTC:], th[:, NTC:], c)
    idx_sc, md_sc = _sc_call(cb, rd.reshape(5 * NSC))

    idx_tc, md_tc = _tc_match(p6[:, :NTC], th[:, :NTC],
                              map_token_sample_pt.reshape(K, 6))

    idx = jnp.concatenate([idx_tc, idx_sc])
    md = jnp.concatenate([md_tc, md_sc])
    return (traj_pos[:, 0], traj_theta, idx, md)


# speedup vs baseline: 1.0398x; 1.0362x over previous
import functools

import jax
import jax.numpy as jnp
from jax import lax
from jax.experimental import pallas as pl
from jax.experimental.pallas import tpu as pltpu
from jax.experimental.pallas import tpu_sc as plsc

N = 16384
K = 2048

NSC = 5120
NTC = N - NSC

NSUB = 32
RP = NSC // NSUB
CH = K // 16
G = 4

BNP = 1024
BNT = 1024


def _prep_body(pt_ref, th_ref, c_ref,
               rd1_ref, rd2_ref, rd3_ref, rd4_ref, rd5_ref, cb_ref):
    pt = pt_ref[...]
    th = th_ref[...]
    cos = jnp.cos(th)
    sin = jnp.sin(th)
    dx1 = pt[2:3, :] - pt[0:1, :]
    dy1 = pt[3:4, :] - pt[1:2, :]
    dx2 = pt[4:5, :] - pt[0:1, :]
    dy2 = pt[5:6, :] - pt[1:2, :]
    px1 = dx1 * cos + dy1 * sin
    py1 = dy1 * cos - dx1 * sin
    px2 = dx2 * cos + dy2 * sin
    py2 = dy2 * cos - dx2 * sin
    pn = dx1 * dx1 + dy1 * dy1 + dx2 * dx2 + dy2 * dy2
    rd_refs = (rd1_ref, rd2_ref, rd3_ref, rd4_ref, rd5_ref)
    for ref, val in zip(rd_refs, (px1, py1, px2, py2, pn)):
        ref[...] = val[0]

    c = c_ref[...]
    cx1 = c[2:3, :]
    cy1 = c[3:4, :]
    cx2 = c[4:5, :]
    cy2 = c[5:6, :]
    e = (c[0:1, :] * c[0:1, :] + c[1:2, :] * c[1:2, :]
         + cx1 * cx1 + cy1 * cy1 + cx2 * cx2 + cy2 * cy2)
    zk3 = jnp.zeros((3, K), jnp.float32)
    cb_ref[...] = jnp.concatenate(
        [2.0 * cx1, 2.0 * cy1, 2.0 * cx2, 2.0 * cy2, e, zk3], axis=0)


def _tc_prep(pt_sc, th_sc, c):
    rdt = jax.ShapeDtypeStruct((NSC,), jnp.float32)
    return pl.pallas_call(
        _prep_body,
        out_shape=[rdt, rdt, rdt, rdt, rdt,
                   jax.ShapeDtypeStruct((8, K), jnp.float32)],
    )(pt_sc, th_sc, c)


def _sc_body(cb_hbm, rd1_hbm, rd2_hbm, rd3_hbm, rd4_hbm, rd5_hbm,
             idx_hbm, md_hbm, cb_v, rd_v, idx_v, md_v):
    wid = lax.axis_index("s") * 2 + lax.axis_index("c")
    base = wid * RP
    pltpu.sync_copy(cb_hbm, cb_v)
    for comp, rd_hbm in enumerate((rd1_hbm, rd2_hbm, rd3_hbm, rd4_hbm, rd5_hbm)):
        pltpu.sync_copy(rd_hbm.at[pl.ds(base, RP)],
                        rd_v.at[pl.ds(comp * RP, RP)])
    kiota = lax.iota(jnp.int32, 16)
    lane0 = kiota == 0

    def macro_body(mb, _):
        rbase = mb * 16
        av1 = rd_v[pl.ds(0 * RP + rbase, 16)]
        av2 = rd_v[pl.ds(1 * RP + rbase, 16)]
        av3 = rd_v[pl.ds(2 * RP + rbase, 16)]
        av4 = rd_v[pl.ds(3 * RP + rbase, 16)]
        apn = rd_v[pl.ds(4 * RP + rbase, 16)]

        for sub in range(16 // G):
            s1, s2, s3, s4 = [], [], [], []
            for i in range(G):
                li = jnp.full((16,), sub * G + i, jnp.int32)
                s1.append(jnp.take_along_axis(av1, li, axis=0))
                s2.append(jnp.take_along_axis(av2, li, axis=0))
                s3.append(jnp.take_along_axis(av3, li, axis=0))
                s4.append(jnp.take_along_axis(av4, li, axis=0))

            def chunk_body(j, carry, s1=s1, s2=s2, s3=s3, s4=s4):
                best, bidx = carry
                o = j * 16
                c1 = cb_v[0, pl.ds(o, 16)]
                c2 = cb_v[1, pl.ds(o, 16)]
                c3 = cb_v[2, pl.ds(o, 16)]
                c4 = cb_v[3, pl.ds(o, 16)]
                ev = cb_v[4, pl.ds(o, 16)]
                jv = jnp.full((16,), j, jnp.int32)
                nbest, nbidx = [], []
                for i in range(G):
                    d = ev - (c1 * s1[i] + c2 * s2[i] + c3 * s3[i] + c4 * s4[i])
                    lt = d < best[i]
                    nbest.append(jnp.where(lt, d, best[i]))
                    nbidx.append(jnp.where(lt, jv, bidx[i]))
                return tuple(nbest), tuple(nbidx)

            best0 = tuple(jnp.full((16,), jnp.inf, jnp.float32) for _ in range(G))
            bidx0 = tuple(jnp.zeros((16,), jnp.int32) for _ in range(G))
            best, bidx = lax.fori_loop(0, CH, chunk_body, (best0, bidx0))

            for i in range(G):
                mv = jnp.min(best[i])
                bi = jnp.min(jnp.where(best[i] == mv,
                                       bidx[i] * 16 + kiota, jnp.int32(K)))
                r = rbase + sub * G + i
                rv = jnp.full((16,), r, jnp.int32)
                plsc.store_scatter(idx_v, [rv], jnp.full((16,), bi, jnp.int32),
                                   mask=lane0)
                plsc.store_scatter(md_v, [rv], jnp.full((16,), mv + apn[sub * G + i],
                                                        jnp.float32), mask=lane0)
        return 0

    lax.fori_loop(0, RP // 16, macro_body, 0)
    pltpu.sync_copy(idx_v, idx_hbm.at[pl.ds(base, RP)])
    pltpu.sync_copy(md_v, md_hbm.at[pl.ds(base, RP)])


_sc_call = functools.partial(
    pl.kernel,
    mesh=plsc.VectorSubcoreMesh(core_axis_name="c", subcore_axis_name="s"),
    compiler_params=pltpu.CompilerParams(needs_layout_passes=False),
    out_type=[
        jax.ShapeDtypeStruct((NSC,), jnp.int32),
        jax.ShapeDtypeStruct((NSC,), jnp.float32),
    ],
    scratch_types=[
        pltpu.VMEM((8, K), jnp.float32),
        pltpu.VMEM((5 * RP,), jnp.float32),
        pltpu.VMEM((RP,), jnp.int32),
        pltpu.VMEM((RP,), jnp.float32),
    ],
)(_sc_body)


KB = 512


def _tc_match_body(pt_ref, th_ref, ct_ref, idx_ref, md_ref):
    pt = pt_ref[...]
    th = th_ref[...]
    cos = jnp.cos(th)
    sin = jnp.sin(th)
    dx1 = pt[2:3, :] - pt[0:1, :]
    dy1 = pt[3:4, :] - pt[1:2, :]
    dx2 = pt[4:5, :] - pt[0:1, :]
    dy2 = pt[5:6, :] - pt[1:2, :]
    px1 = dx1 * cos + dy1 * sin
    py1 = dy1 * cos - dx1 * sin
    px2 = dx2 * cos + dy2 * sin
    py2 = dy2 * cos - dx2 * sin
    pn = dx1 * dx1 + dy1 * dy1 + dx2 * dx2 + dy2 * dy2

    ct = ct_ref[...]
    e_all = jnp.sum(ct * ct, axis=1, keepdims=True)

    m = jnp.full((1, BNT), jnp.inf, jnp.float32)
    am = jnp.full((1, BNT), K, jnp.int32)
    iota0 = lax.broadcasted_iota(jnp.int32, (KB, BNT), 0)
    for b in range(K // KB):
        sl = slice(b * KB, (b + 1) * KB)
        cx1 = ct[sl, 2:3]
        cy1 = ct[sl, 3:4]
        cx2 = ct[sl, 4:5]
        cy2 = ct[sl, 5:6]
        d = e_all[sl] - ((cx1 + cx1) * px1 + (cy1 + cy1) * py1
                         + (cx2 + cx2) * px2 + (cy2 + cy2) * py2)
        mc = jnp.min(d, axis=0, keepdims=True)
        cand = jnp.where(d <= mc, iota0, jnp.int32(K))
        amc = jnp.min(cand, axis=0, keepdims=True) + b * KB
        upd = mc < m
        m = jnp.where(upd, mc, m)
        am = jnp.where(upd, amc, am)

    idx_ref[...] = am
    md_ref[...] = m + pn


def _tc_match(pt_tc, th_tc, ct):
    nb = NTC // BNT
    idx2, md2 = pl.pallas_call(
        _tc_match_body,
        grid=(nb,),
        in_specs=[
            pl.BlockSpec((6, BNT), lambda i: (0, i)),
            pl.BlockSpec((1, BNT), lambda i: (0, i)),
            pl.BlockSpec((K, 6), lambda i: (0, 0)),
        ],
        out_specs=[
            pl.BlockSpec((1, BNT), lambda i: (0, i)),
            pl.BlockSpec((1, BNT), lambda i: (0, i)),
        ],
        out_shape=[
            jax.ShapeDtypeStruct((1, NTC), jnp.int32),
            jax.ShapeDtypeStruct((1, NTC), jnp.float32),
        ],
    )(pt_tc, th_tc, ct)
    return idx2.reshape(NTC), md2.reshape(NTC)


@jax.jit
def kernel(traj_pos, traj_theta, map_token_sample_pt):
    p6 = traj_pos.reshape(N, 6).T
    th = traj_theta.reshape(1, N)
    c = map_token_sample_pt.reshape(K, 6).T

    r1, r2, r3, r4, r5, cb = _tc_prep(p6[:, NTC:], th[:, NTC:], c)
    idx_sc, md_sc = _sc_call(cb, r1, r2, r3, r4, r5)

    idx_tc, md_tc = _tc_match(p6[:, :NTC], th[:, :NTC],
                              map_token_sample_pt.reshape(K, 6))

    idx = jnp.concatenate([idx_tc, idx_sc])
    md = jnp.concatenate([md_tc, md_sc])
    return (traj_pos[:, 0], traj_theta, idx, md)


# speedup vs baseline: 1.0404x; 1.0006x over previous
import functools

import jax
import jax.numpy as jnp
from jax import lax
from jax.experimental import pallas as pl
from jax.experimental.pallas import tpu as pltpu
from jax.experimental.pallas import tpu_sc as plsc

N = 16384
K = 2048

NSC = 5120
NTC = N - NSC

NSUB = 32
RP = NSC // NSUB
CH = K // 16
G = 4

BNP = 1024
BNT = 1024


def _prep_body(pt_ref, th_ref, c_ref,
               rd1_ref, rd2_ref, rd3_ref, rd4_ref, rd5_ref, cb_ref):
    pt = pt_ref[...]
    th = th_ref[...]
    cos = jnp.cos(th)
    sin = jnp.sin(th)
    dx1 = pt[2:3, :] - pt[0:1, :]
    dy1 = pt[3:4, :] - pt[1:2, :]
    dx2 = pt[4:5, :] - pt[0:1, :]
    dy2 = pt[5:6, :] - pt[1:2, :]
    px1 = dx1 * cos + dy1 * sin
    py1 = dy1 * cos - dx1 * sin
    px2 = dx2 * cos + dy2 * sin
    py2 = dy2 * cos - dx2 * sin
    pn = dx1 * dx1 + dy1 * dy1 + dx2 * dx2 + dy2 * dy2
    rd_refs = (rd1_ref, rd2_ref, rd3_ref, rd4_ref, rd5_ref)
    for ref, val in zip(rd_refs, (px1, py1, px2, py2, pn)):
        ref[...] = val[0]

    c = c_ref[...]
    cx1 = c[2:3, :]
    cy1 = c[3:4, :]
    cx2 = c[4:5, :]
    cy2 = c[5:6, :]
    e = (c[0:1, :] * c[0:1, :] + c[1:2, :] * c[1:2, :]
         + cx1 * cx1 + cy1 * cy1 + cx2 * cx2 + cy2 * cy2)
    zk3 = jnp.zeros((3, K), jnp.float32)
    cb_ref[...] = jnp.concatenate(
        [2.0 * cx1, 2.0 * cy1, 2.0 * cx2, 2.0 * cy2, e, zk3], axis=0)


def _tc_prep(pt_sc, th_sc, c):
    rdt = jax.ShapeDtypeStruct((NSC,), jnp.float32)
    return pl.pallas_call(
        _prep_body,
        out_shape=[rdt, rdt, rdt, rdt, rdt,
                   jax.ShapeDtypeStruct((8, K), jnp.float32)],
    )(pt_sc, th_sc, c)


def _sc_body(cb_hbm, rd1_hbm, rd2_hbm, rd3_hbm, rd4_hbm, rd5_hbm,
             idx_hbm, md_hbm, cb_v, rd_v, idx_v, md_v):
    wid = lax.axis_index("s") * 2 + lax.axis_index("c")
    base = wid * RP
    pltpu.sync_copy(cb_hbm, cb_v)
    for comp, rd_hbm in enumerate((rd1_hbm, rd2_hbm, rd3_hbm, rd4_hbm, rd5_hbm)):
        pltpu.sync_copy(rd_hbm.at[pl.ds(base, RP)],
                        rd_v.at[pl.ds(comp * RP, RP)])
    kiota = lax.iota(jnp.int32, 16)
    lane0 = kiota == 0

    def macro_body(mb, _):
        rbase = mb * 16
        av1 = rd_v[pl.ds(0 * RP + rbase, 16)]
        av2 = rd_v[pl.ds(1 * RP + rbase, 16)]
        av3 = rd_v[pl.ds(2 * RP + rbase, 16)]
        av4 = rd_v[pl.ds(3 * RP + rbase, 16)]
        apn = rd_v[pl.ds(4 * RP + rbase, 16)]

        for sub in range(16 // G):
            s1, s2, s3, s4 = [], [], [], []
            for i in range(G):
                li = jnp.full((16,), sub * G + i, jnp.int32)
                s1.append(jnp.take_along_axis(av1, li, axis=0))
                s2.append(jnp.take_along_axis(av2, li, axis=0))
                s3.append(jnp.take_along_axis(av3, li, axis=0))
                s4.append(jnp.take_along_axis(av4, li, axis=0))

            def chunk_body(t, carry, s1=s1, s2=s2, s3=s3, s4=s4):
                bestA, bidxA, bestB, bidxB = carry
                ja = t * 2
                jb = ja + 1
                oa = ja * 16
                ob = oa + 16
                c1a = cb_v[0, pl.ds(oa, 16)]
                c2a = cb_v[1, pl.ds(oa, 16)]
                c3a = cb_v[2, pl.ds(oa, 16)]
                c4a = cb_v[3, pl.ds(oa, 16)]
                eva = cb_v[4, pl.ds(oa, 16)]
                c1b = cb_v[0, pl.ds(ob, 16)]
                c2b = cb_v[1, pl.ds(ob, 16)]
                c3b = cb_v[2, pl.ds(ob, 16)]
                c4b = cb_v[3, pl.ds(ob, 16)]
                evb = cb_v[4, pl.ds(ob, 16)]
                jva = jnp.full((16,), ja, jnp.int32)
                jvb = jnp.full((16,), jb, jnp.int32)
                nbA, niA, nbB, niB = [], [], [], []
                for i in range(G):
                    da = eva - (c1a * s1[i] + c2a * s2[i]
                                + c3a * s3[i] + c4a * s4[i])
                    lta = da < bestA[i]
                    nbA.append(jnp.where(lta, da, bestA[i]))
                    niA.append(jnp.where(lta, jva, bidxA[i]))
                    db = evb - (c1b * s1[i] + c2b * s2[i]
                                + c3b * s3[i] + c4b * s4[i])
                    ltb = db < bestB[i]
                    nbB.append(jnp.where(ltb, db, bestB[i]))
                    niB.append(jnp.where(ltb, jvb, bidxB[i]))
                return tuple(nbA), tuple(niA), tuple(nbB), tuple(niB)

            best0 = tuple(jnp.full((16,), jnp.inf, jnp.float32) for _ in range(G))
            bidx0 = tuple(jnp.zeros((16,), jnp.int32) for _ in range(G))
            bestA, bidxA, bestB, bidxB = lax.fori_loop(
                0, CH // 2, chunk_body, (best0, bidx0, best0, bidx0))

            for i in range(G):
                ltb = bestB[i] < bestA[i]
                eq = bestB[i] == bestA[i]
                best_i = jnp.where(ltb, bestB[i], bestA[i])
                bidx_i = jnp.where(ltb, bidxB[i], bidxA[i])
                bidx_i = jnp.where(eq, jnp.minimum(bidxA[i], bidxB[i]), bidx_i)
                mv = jnp.min(best_i)
                bi = jnp.min(jnp.where(best_i == mv,
                                       bidx_i * 16 + kiota, jnp.int32(K)))
                r = rbase + sub * G + i
                rv = jnp.full((16,), r, jnp.int32)
                plsc.store_scatter(idx_v, [rv], jnp.full((16,), bi, jnp.int32),
                                   mask=lane0)
                plsc.store_scatter(md_v, [rv], jnp.full((16,), mv + apn[sub * G + i],
                                                        jnp.float32), mask=lane0)
        return 0

    lax.fori_loop(0, RP // 16, macro_body, 0)
    pltpu.sync_copy(idx_v, idx_hbm.at[pl.ds(base, RP)])
    pltpu.sync_copy(md_v, md_hbm.at[pl.ds(base, RP)])


_sc_call = functools.partial(
    pl.kernel,
    mesh=plsc.VectorSubcoreMesh(core_axis_name="c", subcore_axis_name="s"),
    compiler_params=pltpu.CompilerParams(needs_layout_passes=False),
    out_type=[
        jax.ShapeDtypeStruct((NSC,), jnp.int32),
        jax.ShapeDtypeStruct((NSC,), jnp.float32),
    ],
    scratch_types=[
        pltpu.VMEM((8, K), jnp.float32),
        pltpu.VMEM((5 * RP,), jnp.float32),
        pltpu.VMEM((RP,), jnp.int32),
        pltpu.VMEM((RP,), jnp.float32),
    ],
)(_sc_body)


KB = 512


def _tc_match_body(pt_ref, th_ref, ct_ref, idx_ref, md_ref):
    pt = pt_ref[...]
    th = th_ref[...]
    cos = jnp.cos(th)
    sin = jnp.sin(th)
    dx1 = pt[2:3, :] - pt[0:1, :]
    dy1 = pt[3:4, :] - pt[1:2, :]
    dx2 = pt[4:5, :] - pt[0:1, :]
    dy2 = pt[5:6, :] - pt[1:2, :]
    px1 = dx1 * cos + dy1 * sin
    py1 = dy1 * cos - dx1 * sin
    px2 = dx2 * cos + dy2 * sin
    py2 = dy2 * cos - dx2 * sin
    pn = dx1 * dx1 + dy1 * dy1 + dx2 * dx2 + dy2 * dy2

    ct = ct_ref[...]
    e_all = jnp.sum(ct * ct, axis=1, keepdims=True)

    m = jnp.full((1, BNT), jnp.inf, jnp.float32)
    am = jnp.full((1, BNT), K, jnp.int32)
    iota0 = lax.broadcasted_iota(jnp.int32, (KB, BNT), 0)
    for b in range(K // KB):
        sl = slice(b * KB, (b + 1) * KB)
        cx1 = ct[sl, 2:3]
        cy1 = ct[sl, 3:4]
        cx2 = ct[sl, 4:5]
        cy2 = ct[sl, 5:6]
        d = e_all[sl] - ((cx1 + cx1) * px1 + (cy1 + cy1) * py1
                         + (cx2 + cx2) * px2 + (cy2 + cy2) * py2)
        mc = jnp.min(d, axis=0, keepdims=True)
        cand = jnp.where(d <= mc, iota0, jnp.int32(K))
        amc = jnp.min(cand, axis=0, keepdims=True) + b * KB
        upd = mc < m
        m = jnp.where(upd, mc, m)
        am = jnp.where(upd, amc, am)

    idx_ref[...] = am
    md_ref[...] = m + pn


def _tc_match(pt_tc, th_tc, ct):
    nb = NTC // BNT
    idx2, md2 = pl.pallas_call(
        _tc_match_body,
        grid=(nb,),
        in_specs=[
            pl.BlockSpec((6, BNT), lambda i: (0, i)),
            pl.BlockSpec((1, BNT), lambda i: (0, i)),
            pl.BlockSpec((K, 6), lambda i: (0, 0)),
        ],
        out_specs=[
            pl.BlockSpec((1, BNT), lambda i: (0, i)),
            pl.BlockSpec((1, BNT), lambda i: (0, i)),
        ],
        out_shape=[
            jax.ShapeDtypeStruct((1, NTC), jnp.int32),
            jax.ShapeDtypeStruct((1, NTC), jnp.float32),
        ],
    )(pt_tc, th_tc, ct)
    return idx2.reshape(NTC), md2.reshape(NTC)


@jax.jit
def kernel(traj_pos, traj_theta, map_token_sample_pt):
    p6 = traj_pos.reshape(N, 6).T
    th = traj_theta.reshape(1, N)
    c = map_token_sample_pt.reshape(K, 6).T

    r1, r2, r3, r4, r5, cb = _tc_prep(p6[:, NTC:], th[:, NTC:], c)
    idx_sc, md_sc = _sc_call(cb, r1, r2, r3, r4, r5)

    idx_tc, md_tc = _tc_match(p6[:, :NTC], th[:, :NTC],
                              map_token_sample_pt.reshape(K, 6))

    idx = jnp.concatenate([idx_tc, idx_sc])
    md = jnp.concatenate([md_tc, md_sc])
    return (traj_pos[:, 0], traj_theta, idx, md)


# speedup vs baseline: 1.0748x; 1.0331x over previous
import functools

import jax
import jax.numpy as jnp
from jax import lax
from jax.experimental import pallas as pl
from jax.experimental.pallas import tpu as pltpu
from jax.experimental.pallas import tpu_sc as plsc

N = 16384
K = 2048

NSC = 5120
NTC = N - NSC

NSUB = 32
RP = NSC // NSUB
CH = K // 16
G = 4

BNP = 1024
BNT = 1024


def _prep_body(pt_ref, th_ref, c_ref,
               rd1_ref, rd2_ref, rd3_ref, rd4_ref, rd5_ref, cb_ref):
    pt = pt_ref[...]
    th = th_ref[...]
    cos = jnp.cos(th)
    sin = jnp.sin(th)
    dx1 = pt[2:3, :] - pt[0:1, :]
    dy1 = pt[3:4, :] - pt[1:2, :]
    dx2 = pt[4:5, :] - pt[0:1, :]
    dy2 = pt[5:6, :] - pt[1:2, :]
    px1 = dx1 * cos + dy1 * sin
    py1 = dy1 * cos - dx1 * sin
    px2 = dx2 * cos + dy2 * sin
    py2 = dy2 * cos - dx2 * sin
    pn = dx1 * dx1 + dy1 * dy1 + dx2 * dx2 + dy2 * dy2
    rd_refs = (rd1_ref, rd2_ref, rd3_ref, rd4_ref, rd5_ref)
    for ref, val in zip(rd_refs, (px1, py1, px2, py2, pn)):
        ref[...] = val[0]

    c = c_ref[...]
    cx1 = c[2:3, :]
    cy1 = c[3:4, :]
    cx2 = c[4:5, :]
    cy2 = c[5:6, :]
    e = (c[0:1, :] * c[0:1, :] + c[1:2, :] * c[1:2, :]
         + cx1 * cx1 + cy1 * cy1 + cx2 * cx2 + cy2 * cy2)
    zk3 = jnp.zeros((3, K), jnp.float32)
    cb_ref[...] = jnp.concatenate(
        [2.0 * cx1, 2.0 * cy1, 2.0 * cx2, 2.0 * cy2, e, zk3], axis=0)


def _tc_prep(p6, th, c):
    rdt = jax.ShapeDtypeStruct((NSC,), jnp.float32)
    return pl.pallas_call(
        _prep_body,
        grid=(1,),
        in_specs=[
            pl.BlockSpec((6, NSC), lambda i: (0, 0)),
            pl.BlockSpec((1, NSC), lambda i: (0, 0)),
            pl.BlockSpec((6, K), lambda i: (0, 0)),
        ],
        out_specs=[pl.BlockSpec((NSC,), lambda i: (0,))] * 5
        + [pl.BlockSpec((8, K), lambda i: (0, 0))],
        out_shape=[rdt, rdt, rdt, rdt, rdt,
                   jax.ShapeDtypeStruct((8, K), jnp.float32)],
    )(p6, th, c)


def _sc_body(cb_hbm, rd1_hbm, rd2_hbm, rd3_hbm, rd4_hbm, rd5_hbm,
             idx_hbm, md_hbm, cb_v, rd_v, idx_v, md_v):
    wid = lax.axis_index("s") * 2 + lax.axis_index("c")
    base = wid * RP
    pltpu.sync_copy(cb_hbm, cb_v)
    for comp, rd_hbm in enumerate((rd1_hbm, rd2_hbm, rd3_hbm, rd4_hbm, rd5_hbm)):
        pltpu.sync_copy(rd_hbm.at[pl.ds(base, RP)],
                        rd_v.at[pl.ds(comp * RP, RP)])
    kiota = lax.iota(jnp.int32, 16)
    lane0 = kiota == 0

    def macro_body(mb, _):
        rbase = mb * 16
        av1 = rd_v[pl.ds(0 * RP + rbase, 16)]
        av2 = rd_v[pl.ds(1 * RP + rbase, 16)]
        av3 = rd_v[pl.ds(2 * RP + rbase, 16)]
        av4 = rd_v[pl.ds(3 * RP + rbase, 16)]
        apn = rd_v[pl.ds(4 * RP + rbase, 16)]

        for sub in range(16 // G):
            s1, s2, s3, s4 = [], [], [], []
            for i in range(G):
                li = jnp.full((16,), sub * G + i, jnp.int32)
                s1.append(jnp.take_along_axis(av1, li, axis=0))
                s2.append(jnp.take_along_axis(av2, li, axis=0))
                s3.append(jnp.take_along_axis(av3, li, axis=0))
                s4.append(jnp.take_along_axis(av4, li, axis=0))

            def chunk_body(t, carry, s1=s1, s2=s2, s3=s3, s4=s4):
                bestA, bidxA, bestB, bidxB = carry
                ja = t * 2
                jb = ja + 1
                oa = ja * 16
                ob = oa + 16
                c1a = cb_v[0, pl.ds(oa, 16)]
                c2a = cb_v[1, pl.ds(oa, 16)]
                c3a = cb_v[2, pl.ds(oa, 16)]
                c4a = cb_v[3, pl.ds(oa, 16)]
                eva = cb_v[4, pl.ds(oa, 16)]
                c1b = cb_v[0, pl.ds(ob, 16)]
                c2b = cb_v[1, pl.ds(ob, 16)]
                c3b = cb_v[2, pl.ds(ob, 16)]
                c4b = cb_v[3, pl.ds(ob, 16)]
                evb = cb_v[4, pl.ds(ob, 16)]
                jva = jnp.full((16,), ja, jnp.int32)
                jvb = jnp.full((16,), jb, jnp.int32)
                nbA, niA, nbB, niB = [], [], [], []
                for i in range(G):
                    da = eva - (c1a * s1[i] + c2a * s2[i]
                                + c3a * s3[i] + c4a * s4[i])
                    lta = da < bestA[i]
                    nbA.append(jnp.where(lta, da, bestA[i]))
                    niA.append(jnp.where(lta, jva, bidxA[i]))
                    db = evb - (c1b * s1[i] + c2b * s2[i]
                                + c3b * s3[i] + c4b * s4[i])
                    ltb = db < bestB[i]
                    nbB.append(jnp.where(ltb, db, bestB[i]))
                    niB.append(jnp.where(ltb, jvb, bidxB[i]))
                return tuple(nbA), tuple(niA), tuple(nbB), tuple(niB)

            best0 = tuple(jnp.full((16,), jnp.inf, jnp.float32) for _ in range(G))
            bidx0 = tuple(jnp.zeros((16,), jnp.int32) for _ in range(G))
            bestA, bidxA, bestB, bidxB = lax.fori_loop(
                0, CH // 2, chunk_body, (best0, bidx0, best0, bidx0))

            for i in range(G):
                ltb = bestB[i] < bestA[i]
                eq = bestB[i] == bestA[i]
                best_i = jnp.where(ltb, bestB[i], bestA[i])
                bidx_i = jnp.where(ltb, bidxB[i], bidxA[i])
                bidx_i = jnp.where(eq, jnp.minimum(bidxA[i], bidxB[i]), bidx_i)
                mv = jnp.min(best_i)
                bi = jnp.min(jnp.where(best_i == mv,
                                       bidx_i * 16 + kiota, jnp.int32(K)))
                r = rbase + sub * G + i
                rv = jnp.full((16,), r, jnp.int32)
                plsc.store_scatter(idx_v, [rv], jnp.full((16,), bi, jnp.int32),
                                   mask=lane0)
                plsc.store_scatter(md_v, [rv], jnp.full((16,), mv + apn[sub * G + i],
                                                        jnp.float32), mask=lane0)
        return 0

    lax.fori_loop(0, RP // 16, macro_body, 0)
    pltpu.sync_copy(idx_v, idx_hbm.at[pl.ds(base, RP)])
    pltpu.sync_copy(md_v, md_hbm.at[pl.ds(base, RP)])


_sc_call = functools.partial(
    pl.kernel,
    mesh=plsc.VectorSubcoreMesh(core_axis_name="c", subcore_axis_name="s"),
    compiler_params=pltpu.CompilerParams(needs_layout_passes=False),
    out_type=[
        jax.ShapeDtypeStruct((NSC,), jnp.int32),
        jax.ShapeDtypeStruct((NSC,), jnp.float32),
    ],
    scratch_types=[
        pltpu.VMEM((8, K), jnp.float32),
        pltpu.VMEM((5 * RP,), jnp.float32),
        pltpu.VMEM((RP,), jnp.int32),
        pltpu.VMEM((RP,), jnp.float32),
    ],
)(_sc_body)


KB = 512


def _tc_match_body(pt_ref, th_ref, ct_ref, idx_ref, md_ref):
    pt = pt_ref[...]
    th = th_ref[...]
    cos = jnp.cos(th)
    sin = jnp.sin(th)
    dx1 = pt[2:3, :] - pt[0:1, :]
    dy1 = pt[3:4, :] - pt[1:2, :]
    dx2 = pt[4:5, :] - pt[0:1, :]
    dy2 = pt[5:6, :] - pt[1:2, :]
    px1 = dx1 * cos + dy1 * sin
    py1 = dy1 * cos - dx1 * sin
    px2 = dx2 * cos + dy2 * sin
    py2 = dy2 * cos - dx2 * sin
    pn = dx1 * dx1 + dy1 * dy1 + dx2 * dx2 + dy2 * dy2

    ct = ct_ref[...]
    e_all = jnp.sum(ct * ct, axis=1, keepdims=True)

    m = jnp.full((1, BNT), jnp.inf, jnp.float32)
    am = jnp.full((1, BNT), K, jnp.int32)
    iota0 = lax.broadcasted_iota(jnp.int32, (KB, BNT), 0)
    for b in range(K // KB):
        sl = slice(b * KB, (b + 1) * KB)
        cx1 = ct[sl, 2:3]
        cy1 = ct[sl, 3:4]
        cx2 = ct[sl, 4:5]
        cy2 = ct[sl, 5:6]
        d = e_all[sl] - ((cx1 + cx1) * px1 + (cy1 + cy1) * py1
                         + (cx2 + cx2) * px2 + (cy2 + cy2) * py2)
        mc = jnp.min(d, axis=0, keepdims=True)
        cand = jnp.where(d <= mc, iota0, jnp.int32(K))
        amc = jnp.min(cand, axis=0, keepdims=True) + b * KB
        upd = mc < m
        m = jnp.where(upd, mc, m)
        am = jnp.where(upd, amc, am)

    idx_ref[...] = am
    md_ref[...] = m + pn


def _tc_match(p6, th, ct):
    off = NSC // BNT
    nb = NTC // BNT
    idx2, md2 = pl.pallas_call(
        _tc_match_body,
        grid=(nb,),
        in_specs=[
            pl.BlockSpec((6, BNT), lambda i: (0, i + off)),
            pl.BlockSpec((1, BNT), lambda i: (0, i + off)),
            pl.BlockSpec((K, 6), lambda i: (0, 0)),
        ],
        out_specs=[
            pl.BlockSpec((1, BNT), lambda i: (0, i)),
            pl.BlockSpec((1, BNT), lambda i: (0, i)),
        ],
        out_shape=[
            jax.ShapeDtypeStruct((1, NTC), jnp.int32),
            jax.ShapeDtypeStruct((1, NTC), jnp.float32),
        ],
    )(p6, th, ct)
    return idx2.reshape(NTC), md2.reshape(NTC)


@jax.jit
def kernel(traj_pos, traj_theta, map_token_sample_pt):
    p6 = traj_pos.reshape(N, 6).T
    th = traj_theta.reshape(1, N)
    c = map_token_sample_pt.reshape(K, 6).T

    r1, r2, r3, r4, r5, cb = _tc_prep(p6, th, c)
    idx_sc, md_sc = _sc_call(cb, r1, r2, r3, r4, r5)

    idx_tc, md_tc = _tc_match(p6, th, map_token_sample_pt.reshape(K, 6))

    idx = jnp.concatenate([idx_sc, idx_tc])
    md = jnp.concatenate([md_sc, md_tc])
    return (traj_pos[:, 0], traj_theta, idx, md)
